# baseline (speedup 1.0000x reference)
import functools

import jax
import jax.numpy as jnp
from jax import lax
from jax.experimental import pallas as pl
from jax.experimental.pallas import tpu as pltpu
from jax.experimental.pallas import tpu_sc as plsc

H = 224
W = 224
C = 96
NO = 9
F = 768
NPIX = H * W
NSAMP = NPIX * NO

NC = 2
NS = 16
NW = NC * NS
SW = NSAMP // NW
NB = 48
NBAT = SW // NB


_ROWS_PER_BLK = 8
_NBLK = H // _ROWS_PER_BLK


def _off_conv_body(x_ref, w_ref, b_ref, out_ref, pad_ref):
    r = pl.program_id(0)
    y0 = r * _ROWS_PER_BLK
    nr = _ROWS_PER_BLK + 2
    pad_ref[0, :, :] = jnp.zeros((W + 2, C), jnp.float32)
    pad_ref[nr - 1, :, :] = jnp.zeros((W + 2, C), jnp.float32)
    pad_ref[:, pl.ds(0, 1), :] = jnp.zeros((nr, 1, C), jnp.float32)
    pad_ref[:, pl.ds(W + 1, 1), :] = jnp.zeros((nr, 1, C), jnp.float32)

    @pl.when(r == 0)
    def _():
        pad_ref[pl.ds(1, nr - 1), pl.ds(1, W), :] = x_ref[pl.ds(0, nr - 1),
                                                          :, :]

    @pl.when(r == _NBLK - 1)
    def _():
        pad_ref[pl.ds(0, nr - 1), pl.ds(1, W), :] = x_ref[
            pl.ds(y0 - 1, nr - 1), :, :]

    @pl.when(jnp.logical_and(r > 0, r < _NBLK - 1))
    def _():
        pad_ref[:, pl.ds(1, W), :] = x_ref[pl.ds(y0 - 1, nr), :, :]

    acc = jnp.broadcast_to(b_ref[...], (_ROWS_PER_BLK * W, 2 * NO))
    for dy in range(3):
        for dx in range(3):
            slab = pad_ref[pl.ds(dy, _ROWS_PER_BLK), pl.ds(dx, W), :]
            a = slab.reshape(_ROWS_PER_BLK * W, C)
            acc = acc + jnp.dot(a, w_ref[dy, dx],
                                preferred_element_type=jnp.float32)
    out_ref[...] = acc


def _off_conv(x, w_off, b_off2d):
    return pl.pallas_call(
        _off_conv_body,
        grid=(_NBLK,),
        in_specs=[
            pl.BlockSpec((H, W, C), lambda i: (0, 0, 0)),
            pl.BlockSpec((3, 3, C, 2 * NO), lambda i: (0, 0, 0, 0)),
            pl.BlockSpec((1, 2 * NO), lambda i: (0, 0)),
        ],
        out_specs=pl.BlockSpec((_ROWS_PER_BLK * W, 2 * NO), lambda i: (i, 0)),
        out_shape=jax.ShapeDtypeStruct((NPIX, 2 * NO), jnp.float32),
        scratch_shapes=[
            pltpu.VMEM((_ROWS_PER_BLK + 2, W + 2, C), jnp.float32)],
    )(x, w_off, b_off2d)


_NPAIR = NBAT // 2


def _sc_sample_body(off_hbm, tab_hbm, out_hbm,
                    off_v, ia_v, ib_v, ic_v, id_v,
                    wa_v, wb_v, wc_v, wd_v,
                    ga_v, gb_v, gc_v, gd_v, o_v,
                    semg0, semg1, semo):
    wid = lax.axis_index("s") * NC + lax.axis_index("c")
    lanes = lax.broadcasted_iota(jnp.int32, (16,), 0)
    fmax = jnp.float32(W - 1)
    semg = (semg0, semg1)

    pltpu.sync_copy(off_hbm.at[pl.ds(wid * (SW * 2), SW * 2)], off_v)

    def phase1(t, p):
        for g in range(NB // 16):
            row = t * NB + g * 16 + lanes
            s_vec = wid * SW + row
            pp = lax.div(s_vec, jnp.int32(NO))
            yy = lax.div(pp, jnp.int32(W))
            xx = pp - yy * W
            offx = plsc.load_gather(off_v, [row * 2])
            offy = plsc.load_gather(off_v, [row * 2 + 1])
            new_x = jnp.minimum(jnp.maximum(xx.astype(jnp.float32) + offx,
                                            0.0), fmax)
            new_y = jnp.minimum(jnp.maximum(yy.astype(jnp.float32) + offy,
                                            0.0), fmax)
            x0i = new_x.astype(jnp.int32)
            y0i = new_y.astype(jnp.int32)
            x0f = x0i.astype(jnp.float32)
            y0f = y0i.astype(jnp.float32)
            x1f = jnp.minimum(x0f + 1.0, fmax)
            y1f = jnp.minimum(y0f + 1.0, fmax)
            x1i = x1f.astype(jnp.int32)
            y1i = y1f.astype(jnp.int32)
            u0 = x1f - new_x
            u1 = new_x - x0f
            v0 = y1f - new_y
            v1 = new_y - y0f
            sl = pl.ds(g * 16, 16)
            fl = pl.ds(p * NB + g * 16, 16)
            ia_v[p, sl] = y0i * W + x0i
            ib_v[p, sl] = y1i * W + x0i
            ic_v[p, sl] = y0i * W + x1i
            id_v[p, sl] = y1i * W + x1i
            wa_v[fl] = u0 * v0
            wb_v[fl] = u0 * v1
            wc_v[fl] = u1 * v0
            wd_v[fl] = u1 * v1
        pltpu.async_copy(tab_hbm.at[ia_v.at[p]], ga_v.at[p], semg[p])
        pltpu.async_copy(tab_hbm.at[ib_v.at[p]], gb_v.at[p], semg[p])
        pltpu.async_copy(tab_hbm.at[ic_v.at[p]], gc_v.at[p], semg[p])
        pltpu.async_copy(tab_hbm.at[id_v.at[p]], gd_v.at[p], semg[p])

    def drain_gathers(p):
        for gv in (ga_v, gb_v, gc_v, gd_v):
            pltpu.make_async_copy(tab_hbm.at[pl.ds(0, NB), :], gv.at[p],
                                  semg[p]).wait()

    def drain_out(p):
        pltpu.make_async_copy(o_v.at[p], out_hbm.at[pl.ds(0, NB), :],
                              semo).wait()

    def compute(t, p):
        @plsc.parallel_loop(0, NB, 1, unroll=4)
        def _(i):
            iv = jnp.zeros((16,), jnp.int32) + (p * NB + i)
            wa = plsc.load_gather(wa_v, [iv])
            wb = plsc.load_gather(wb_v, [iv])
            wc = plsc.load_gather(wc_v, [iv])
            wd = plsc.load_gather(wd_v, [iv])
            for j in range(C // 32):
                cs = pl.ds(j * 32, 32)
                a0, a1 = plsc.unpack(ga_v[p, i, cs],
                                     format=plsc.PackFormat.INTERLEAVED)
                b0, b1 = plsc.unpack(gb_v[p, i, cs],
                                     format=plsc.PackFormat.INTERLEAVED)
                c0, c1 = plsc.unpack(gc_v[p, i, cs],
                                     format=plsc.PackFormat.INTERLEAVED)
                d0, d1 = plsc.unpack(gd_v[p, i, cs],
                                     format=plsc.PackFormat.INTERLEAVED)
                acc0 = wa * a0 + wb * b0 + wc * c0 + wd * d0
                acc1 = wa * a1 + wb * b1 + wc * c1 + wd * d1
                o_v[p, i, cs] = plsc.pack(
                    acc0, acc1, format=plsc.PackFormat.INTERLEAVED)
        pltpu.async_copy(o_v.at[p],
                         out_hbm.at[pl.ds(wid * SW + t * NB, NB), :], semo)

    phase1(0, 0)

    def pair_body(q, carry):
        t_a = 2 * q
        phase1(t_a + 1, 1)
        drain_gathers(0)

        @pl.when(q > 0)
        def _():
            drain_out(0)
        compute(t_a, 0)

        @pl.when(q < _NPAIR - 1)
        def _():
            phase1(t_a + 2, 0)
        drain_gathers(1)

        @pl.when(q > 0)
        def _():
            drain_out(1)
        compute(t_a + 1, 1)
        return carry

    lax.fori_loop(0, _NPAIR, pair_body, 0)
    drain_out(0)
    drain_out(1)


def _sc_sample(off_pairs, table):
    mesh = plsc.VectorSubcoreMesh(core_axis_name="c", subcore_axis_name="s")
    fn = functools.partial(
        pl.kernel,
        mesh=mesh,
        compiler_params=pltpu.CompilerParams(needs_layout_passes=False,
                                             use_tc_tiling_on_sc=False),
        out_type=jax.ShapeDtypeStruct((NSAMP, C), jnp.bfloat16),
        scratch_types=[
            pltpu.VMEM((SW * 2,), jnp.float32),
            pltpu.VMEM((2, NB), jnp.int32),
            pltpu.VMEM((2, NB), jnp.int32),
            pltpu.VMEM((2, NB), jnp.int32),
            pltpu.VMEM((2, NB), jnp.int32),
            pltpu.VMEM((2 * NB,), jnp.float32),
            pltpu.VMEM((2 * NB,), jnp.float32),
            pltpu.VMEM((2 * NB,), jnp.float32),
            pltpu.VMEM((2 * NB,), jnp.float32),
            pltpu.VMEM((2, NB, C), jnp.bfloat16),
            pltpu.VMEM((2, NB, C), jnp.bfloat16),
            pltpu.VMEM((2, NB, C), jnp.bfloat16),
            pltpu.VMEM((2, NB, C), jnp.bfloat16),
            pltpu.VMEM((2, NB, C), jnp.bfloat16),
            pltpu.SemaphoreType.DMA,
            pltpu.SemaphoreType.DMA,
            pltpu.SemaphoreType.DMA,
        ],
    )(_sc_sample_body)
    return fn(off_pairs, table)


_M_BLK = 512


def _mm_body(a_ref, w_ref, b_ref, o_ref):
    a_bf = a_ref[...].astype(jnp.bfloat16)
    o_ref[...] = (jnp.dot(a_bf, w_ref[...],
                          preferred_element_type=jnp.float32) + b_ref[...])


def _matmul(a, w, b2d):
    m = a.shape[0]
    k = a.shape[1]
    return pl.pallas_call(
        _mm_body,
        grid=(m // _M_BLK,),
        in_specs=[
            pl.BlockSpec((_M_BLK, k), lambda i: (i, 0)),
            pl.BlockSpec((k, F), lambda i: (0, 0)),
            pl.BlockSpec((1, F), lambda i: (0, 0)),
        ],
        out_specs=pl.BlockSpec((_M_BLK, F), lambda i: (i, 0)),
        out_shape=jax.ShapeDtypeStruct((m, F), jnp.float32),
    )(a, w, b2d)


def kernel(inputs, W_off, b_off, W_pt, b_pt):
    x = inputs[0]
    offs = _off_conv(x, W_off, b_off.reshape(1, 2 * NO))
    off_pairs = offs.reshape(NSAMP * 2)
    table = x.reshape(NPIX, C).astype(jnp.bfloat16)
    sampled = _sc_sample(off_pairs, table)
    a = sampled.reshape(NPIX, NO * C)
    w = W_pt.reshape(NO * C, F).astype(jnp.bfloat16)
    out = _matmul(a, w, b_pt.reshape(1, F))
    return out.reshape(1, H, W, F)


# speedup vs baseline: 1.0438x; 1.0438x over previous
import functools

import jax
import jax.numpy as jnp
from jax import lax
from jax.experimental import pallas as pl
from jax.experimental.pallas import tpu as pltpu
from jax.experimental.pallas import tpu_sc as plsc

H = 224
W = 224
C = 96
NO = 9
F = 768
NPIX = H * W
NSAMP = NPIX * NO

NC = 2
NS = 16
NW = NC * NS
SW = NSAMP // NW
NB = 48
NBAT = SW // NB


_ROWS_PER_BLK = 8
_NBLK = H // _ROWS_PER_BLK


def _off_conv_body(x_ref, w_ref, b_ref, out_ref, tab_ref, pad_ref):
    r = pl.program_id(0)
    y0 = r * _ROWS_PER_BLK
    nr = _ROWS_PER_BLK + 2
    pad_ref[0, :, :] = jnp.zeros((W + 2, C), jnp.float32)
    pad_ref[nr - 1, :, :] = jnp.zeros((W + 2, C), jnp.float32)
    pad_ref[:, pl.ds(0, 1), :] = jnp.zeros((nr, 1, C), jnp.float32)
    pad_ref[:, pl.ds(W + 1, 1), :] = jnp.zeros((nr, 1, C), jnp.float32)

    @pl.when(r == 0)
    def _():
        pad_ref[pl.ds(1, nr - 1), pl.ds(1, W), :] = x_ref[pl.ds(0, nr - 1),
                                                          :, :]

    @pl.when(r == _NBLK - 1)
    def _():
        pad_ref[pl.ds(0, nr - 1), pl.ds(1, W), :] = x_ref[
            pl.ds(y0 - 1, nr - 1), :, :]

    @pl.when(jnp.logical_and(r > 0, r < _NBLK - 1))
    def _():
        pad_ref[:, pl.ds(1, W), :] = x_ref[pl.ds(y0 - 1, nr), :, :]

    acc = jnp.broadcast_to(b_ref[...], (_ROWS_PER_BLK * W, 2 * NO))
    for dy in range(3):
        for dx in range(3):
            slab = pad_ref[pl.ds(dy, _ROWS_PER_BLK), pl.ds(dx, W), :]
            a = slab.reshape(_ROWS_PER_BLK * W, C)
            acc = acc + jnp.dot(a, w_ref[dy, dx],
                                preferred_element_type=jnp.float32)
    out_ref[...] = acc
    center = pad_ref[pl.ds(1, _ROWS_PER_BLK), pl.ds(1, W), :]
    tab_ref[...] = jnp.concatenate(
        [center.reshape(_ROWS_PER_BLK * W, C),
         jnp.zeros((_ROWS_PER_BLK * W, 128 - C), jnp.float32)], axis=1)


def _off_conv(x, w_off, b_off2d):
    return pl.pallas_call(
        _off_conv_body,
        grid=(_NBLK,),
        in_specs=[
            pl.BlockSpec((H, W, C), lambda i: (0, 0, 0)),
            pl.BlockSpec((3, 3, C, 2 * NO), lambda i: (0, 0, 0, 0)),
            pl.BlockSpec((1, 2 * NO), lambda i: (0, 0)),
        ],
        out_specs=[
            pl.BlockSpec((_ROWS_PER_BLK * W, 2 * NO), lambda i: (i, 0)),
            pl.BlockSpec((_ROWS_PER_BLK * W, 128), lambda i: (i, 0)),
        ],
        out_shape=[
            jax.ShapeDtypeStruct((NPIX, 2 * NO), jnp.float32),
            jax.ShapeDtypeStruct((NPIX, 128), jnp.float32),
        ],
        scratch_shapes=[
            pltpu.VMEM((_ROWS_PER_BLK + 2, W + 2, C), jnp.float32)],
    )(x, w_off, b_off2d)


_NPAIR = NBAT // 2


def _sc_sample_body(off_hbm, tab_hbm, out_hbm,
                    off_v, ia_v, ib_v, ic_v, id_v,
                    wa_v, wb_v, wc_v, wd_v,
                    ga_v, gb_v, gc_v, gd_v, o_v,
                    semg0, semg1, semo):
    wid = lax.axis_index("s") * NC + lax.axis_index("c")
    lanes = lax.broadcasted_iota(jnp.int32, (16,), 0)
    fmax = jnp.float32(W - 1)
    semg = (semg0, semg1)

    pltpu.sync_copy(off_hbm.at[pl.ds(wid * (SW * 2), SW * 2)], off_v)

    def phase1(t, p):
        for g in range(NB // 16):
            row = t * NB + g * 16 + lanes
            s_vec = wid * SW + row
            pp = lax.div(s_vec, jnp.int32(NO))
            yy = lax.div(pp, jnp.int32(W))
            xx = pp - yy * W
            offx = plsc.load_gather(off_v, [row * 2])
            offy = plsc.load_gather(off_v, [row * 2 + 1])
            new_x = jnp.minimum(jnp.maximum(xx.astype(jnp.float32) + offx,
                                            0.0), fmax)
            new_y = jnp.minimum(jnp.maximum(yy.astype(jnp.float32) + offy,
                                            0.0), fmax)
            x0i = new_x.astype(jnp.int32)
            y0i = new_y.astype(jnp.int32)
            x0f = x0i.astype(jnp.float32)
            y0f = y0i.astype(jnp.float32)
            x1f = jnp.minimum(x0f + 1.0, fmax)
            y1f = jnp.minimum(y0f + 1.0, fmax)
            x1i = x1f.astype(jnp.int32)
            y1i = y1f.astype(jnp.int32)
            u0 = x1f - new_x
            u1 = new_x - x0f
            v0 = y1f - new_y
            v1 = new_y - y0f
            sl = pl.ds(g * 16, 16)
            fl = pl.ds(p * NB + g * 16, 16)
            ia_v[p, sl] = y0i * W + x0i
            ib_v[p, sl] = y1i * W + x0i
            ic_v[p, sl] = y0i * W + x1i
            id_v[p, sl] = y1i * W + x1i
            wa_v[fl] = u0 * v0
            wb_v[fl] = u0 * v1
            wc_v[fl] = u1 * v0
            wd_v[fl] = u1 * v1
        pltpu.async_copy(tab_hbm.at[ia_v.at[p]], ga_v.at[p], semg[p])
        pltpu.async_copy(tab_hbm.at[ib_v.at[p]], gb_v.at[p], semg[p])
        pltpu.async_copy(tab_hbm.at[ic_v.at[p]], gc_v.at[p], semg[p])
        pltpu.async_copy(tab_hbm.at[id_v.at[p]], gd_v.at[p], semg[p])

    def drain_gathers(p):
        for gv in (ga_v, gb_v, gc_v, gd_v):
            pltpu.make_async_copy(tab_hbm.at[pl.ds(0, NB), :], gv.at[p],
                                  semg[p]).wait()

    def drain_out(p):
        pltpu.make_async_copy(o_v.at[p], out_hbm.at[pl.ds(0, NB), :],
                              semo).wait()

    def compute(t, p):
        @plsc.parallel_loop(0, NB, 1, unroll=4)
        def _(i):
            iv = jnp.zeros((16,), jnp.int32) + (p * NB + i)
            wa = plsc.load_gather(wa_v, [iv])
            wb = plsc.load_gather(wb_v, [iv])
            wc = plsc.load_gather(wc_v, [iv])
            wd = plsc.load_gather(wd_v, [iv])
            for j in range(C // 16):
                cs = pl.ds(j * 16, 16)
                o_v[p, i, cs] = (wa * ga_v[p, i, cs] + wb * gb_v[p, i, cs]
                                 + wc * gc_v[p, i, cs] + wd * gd_v[p, i, cs])
            zf = jnp.zeros((16,), jnp.float32)
            o_v[p, i, pl.ds(C, 16)] = zf
            o_v[p, i, pl.ds(C + 16, 16)] = zf
        pltpu.async_copy(o_v.at[p],
                         out_hbm.at[pl.ds(wid * SW + t * NB, NB), :], semo)

    phase1(0, 0)

    def pair_body(q, carry):
        t_a = 2 * q
        phase1(t_a + 1, 1)
        drain_gathers(0)

        @pl.when(q > 0)
        def _():
            drain_out(0)
        compute(t_a, 0)

        @pl.when(q < _NPAIR - 1)
        def _():
            phase1(t_a + 2, 0)
        drain_gathers(1)

        @pl.when(q > 0)
        def _():
            drain_out(1)
        compute(t_a + 1, 1)
        return carry

    lax.fori_loop(0, _NPAIR, pair_body, 0)
    drain_out(0)
    drain_out(1)


def _sc_sample(off_pairs, table):
    mesh = plsc.VectorSubcoreMesh(core_axis_name="c", subcore_axis_name="s")
    fn = functools.partial(
        pl.kernel,
        mesh=mesh,
        compiler_params=pltpu.CompilerParams(needs_layout_passes=False,
                                             use_tc_tiling_on_sc=True),
        out_type=jax.ShapeDtypeStruct((NSAMP, 128), jnp.float32),
        scratch_types=[
            pltpu.VMEM((SW * 2,), jnp.float32),
            pltpu.VMEM((2, NB), jnp.int32),
            pltpu.VMEM((2, NB), jnp.int32),
            pltpu.VMEM((2, NB), jnp.int32),
            pltpu.VMEM((2, NB), jnp.int32),
            pltpu.VMEM((2 * NB,), jnp.float32),
            pltpu.VMEM((2 * NB,), jnp.float32),
            pltpu.VMEM((2 * NB,), jnp.float32),
            pltpu.VMEM((2 * NB,), jnp.float32),
            pltpu.VMEM((2, NB, 128), jnp.float32),
            pltpu.VMEM((2, NB, 128), jnp.float32),
            pltpu.VMEM((2, NB, 128), jnp.float32),
            pltpu.VMEM((2, NB, 128), jnp.float32),
            pltpu.VMEM((2, NB, 128), jnp.float32),
            pltpu.SemaphoreType.DMA,
            pltpu.SemaphoreType.DMA,
            pltpu.SemaphoreType.DMA,
        ],
    )(_sc_sample_body)
    return fn(off_pairs, table)


_M_BLK = 512


def _mm_body(a_ref, w_ref, b_ref, o_ref):
    a_bf = a_ref[...].astype(jnp.bfloat16).reshape(_M_BLK, NO * 128)
    o_ref[...] = (jnp.dot(a_bf, w_ref[...],
                          preferred_element_type=jnp.float32) + b_ref[...])


def _matmul(a, w, b2d):
    return pl.pallas_call(
        _mm_body,
        grid=(NPIX // _M_BLK,),
        in_specs=[
            pl.BlockSpec((_M_BLK * NO, 128), lambda i: (i, 0)),
            pl.BlockSpec((NO * 128, F), lambda i: (0, 0)),
            pl.BlockSpec((1, F), lambda i: (0, 0)),
        ],
        out_specs=pl.BlockSpec((_M_BLK, F), lambda i: (i, 0)),
        out_shape=jax.ShapeDtypeStruct((NPIX, F), jnp.float32),
    )(a, w, b2d)


def kernel(inputs, W_off, b_off, W_pt, b_pt):
    x = inputs[0]
    offs, table = _off_conv(x, W_off, b_off.reshape(1, 2 * NO))
    off_pairs = offs.reshape(NSAMP * 2)
    sampled = _sc_sample(off_pairs, table)
    wp = jnp.zeros((NO, 128, F), jnp.float32)
    wp = wp.at[:, :C, :].set(W_pt.reshape(NO, C, F))
    w = wp.reshape(NO * 128, F).astype(jnp.bfloat16)
    out = _matmul(sampled, w, b_pt.reshape(1, F))
    return out.reshape(1, H, W, F)


# speedup vs baseline: 1.1003x; 1.0542x over previous
import functools

import jax
import jax.numpy as jnp
from jax import lax
from jax.experimental import pallas as pl
from jax.experimental.pallas import tpu as pltpu
from jax.experimental.pallas import tpu_sc as plsc

H = 224
W = 224
C = 96
NO = 9
F = 768
NPIX = H * W
NSAMP = NPIX * NO

NC = 2
NS = 16
NW = NC * NS
SW = NSAMP // NW
NB = 48
NBAT = SW // NB


_ROWS_PER_BLK = 8
_NBLK = H // _ROWS_PER_BLK


def _off_conv_body(x_ref, w_ref, b_ref, out_ref, tab_ref, pad_ref):
    r = pl.program_id(0)
    y0 = r * _ROWS_PER_BLK
    nr = _ROWS_PER_BLK + 2
    pad_ref[0, :, :] = jnp.zeros((W + 2, C), jnp.float32)
    pad_ref[nr - 1, :, :] = jnp.zeros((W + 2, C), jnp.float32)
    pad_ref[:, pl.ds(0, 1), :] = jnp.zeros((nr, 1, C), jnp.float32)
    pad_ref[:, pl.ds(W + 1, 1), :] = jnp.zeros((nr, 1, C), jnp.float32)

    @pl.when(r == 0)
    def _():
        pad_ref[pl.ds(1, nr - 1), pl.ds(1, W), :] = x_ref[0, pl.ds(0, nr - 1),
                                                          :, :]

    @pl.when(r == _NBLK - 1)
    def _():
        pad_ref[pl.ds(0, nr - 1), pl.ds(1, W), :] = x_ref[
            0, pl.ds(y0 - 1, nr - 1), :, :]

    @pl.when(jnp.logical_and(r > 0, r < _NBLK - 1))
    def _():
        pad_ref[:, pl.ds(1, W), :] = x_ref[0, pl.ds(y0 - 1, nr), :, :]

    acc = jnp.broadcast_to(b_ref[...], (_ROWS_PER_BLK * W, 2 * NO))
    for dy in range(3):
        for dx in range(3):
            slab = pad_ref[pl.ds(dy, _ROWS_PER_BLK), pl.ds(dx, W), :]
            a = slab.reshape(_ROWS_PER_BLK * W, C)
            acc = acc + jnp.dot(a, w_ref[dy, dx],
                                preferred_element_type=jnp.float32)
    out_ref[...] = acc
    center = pad_ref[pl.ds(1, _ROWS_PER_BLK), pl.ds(1, W), :]
    tab_ref[...] = jnp.concatenate(
        [center.reshape(_ROWS_PER_BLK * W, C),
         jnp.zeros((_ROWS_PER_BLK * W, 128 - C), jnp.float32)], axis=1)


def _off_conv(x, w_off, b_off2d):
    return pl.pallas_call(
        _off_conv_body,
        grid=(_NBLK,),
        in_specs=[
            pl.BlockSpec((1, H, W, C), lambda i: (0, 0, 0, 0)),
            pl.BlockSpec((3, 3, C, 2 * NO), lambda i: (0, 0, 0, 0)),
            pl.BlockSpec((1, 2 * NO), lambda i: (0, 0)),
        ],
        out_specs=[
            pl.BlockSpec((_ROWS_PER_BLK * W, 2 * NO), lambda i: (i, 0)),
            pl.BlockSpec((_ROWS_PER_BLK * W, 128), lambda i: (i, 0)),
        ],
        out_shape=[
            jax.ShapeDtypeStruct((NPIX, 2 * NO), jnp.float32),
            jax.ShapeDtypeStruct((NPIX, 128), jnp.float32),
        ],
        scratch_shapes=[
            pltpu.VMEM((_ROWS_PER_BLK + 2, W + 2, C), jnp.float32)],
    )(x, w_off, b_off2d)


_NPAIR = NBAT // 2


def _sc_sample_body(off_hbm, tab_hbm, out_hbm,
                    off_v, ia_v, ib_v, ic_v, id_v,
                    wa_v, wb_v, wc_v, wd_v,
                    ga_v, gb_v, gc_v, gd_v, o_v,
                    semg0, semg1, semo):
    wid = lax.axis_index("s") * NC + lax.axis_index("c")
    lanes = lax.broadcasted_iota(jnp.int32, (16,), 0)
    fmax = jnp.float32(W - 1)
    semg = (semg0, semg1)

    pltpu.sync_copy(off_hbm.at[pl.ds(wid * (SW * 2), SW * 2)], off_v)

    def phase1(t, p):
        for g in range(NB // 16):
            row = t * NB + g * 16 + lanes
            s_vec = wid * SW + row
            pp = lax.div(s_vec, jnp.int32(NO))
            yy = lax.div(pp, jnp.int32(W))
            xx = pp - yy * W
            offx = plsc.load_gather(off_v, [row * 2])
            offy = plsc.load_gather(off_v, [row * 2 + 1])
            new_x = jnp.minimum(jnp.maximum(xx.astype(jnp.float32) + offx,
                                            0.0), fmax)
            new_y = jnp.minimum(jnp.maximum(yy.astype(jnp.float32) + offy,
                                            0.0), fmax)
            x0i = new_x.astype(jnp.int32)
            y0i = new_y.astype(jnp.int32)
            x0f = x0i.astype(jnp.float32)
            y0f = y0i.astype(jnp.float32)
            x1f = jnp.minimum(x0f + 1.0, fmax)
            y1f = jnp.minimum(y0f + 1.0, fmax)
            x1i = x1f.astype(jnp.int32)
            y1i = y1f.astype(jnp.int32)
            u0 = x1f - new_x
            u1 = new_x - x0f
            v0 = y1f - new_y
            v1 = new_y - y0f
            sl = pl.ds(g * 16, 16)
            fl = pl.ds(p * NB + g * 16, 16)
            ia_v[p, sl] = y0i * W + x0i
            ib_v[p, sl] = y1i * W + x0i
            ic_v[p, sl] = y0i * W + x1i
            id_v[p, sl] = y1i * W + x1i
            wa_v[fl] = u0 * v0
            wb_v[fl] = u0 * v1
            wc_v[fl] = u1 * v0
            wd_v[fl] = u1 * v1
        pltpu.async_copy(tab_hbm.at[ia_v.at[p]], ga_v.at[p], semg[p])
        pltpu.async_copy(tab_hbm.at[ib_v.at[p]], gb_v.at[p], semg[p])
        pltpu.async_copy(tab_hbm.at[ic_v.at[p]], gc_v.at[p], semg[p])
        pltpu.async_copy(tab_hbm.at[id_v.at[p]], gd_v.at[p], semg[p])

    def drain_gathers(p):
        for gv in (ga_v, gb_v, gc_v, gd_v):
            pltpu.make_async_copy(tab_hbm.at[pl.ds(0, NB), :], gv.at[p],
                                  semg[p]).wait()

    def drain_out(p):
        pltpu.make_async_copy(o_v.at[p], out_hbm.at[pl.ds(0, NB), :],
                              semo).wait()

    def compute(t, p):
        @plsc.parallel_loop(0, NB, 1, unroll=4)
        def _(i):
            iv = jnp.zeros((16,), jnp.int32) + (p * NB + i)
            wa = plsc.load_gather(wa_v, [iv])
            wb = plsc.load_gather(wb_v, [iv])
            wc = plsc.load_gather(wc_v, [iv])
            wd = plsc.load_gather(wd_v, [iv])
            for j in range(C // 16):
                cs = pl.ds(j * 16, 16)
                o_v[p, i, cs] = (wa * ga_v[p, i, cs] + wb * gb_v[p, i, cs]
                                 + wc * gc_v[p, i, cs] + wd * gd_v[p, i, cs])
            zf = jnp.zeros((16,), jnp.float32)
            o_v[p, i, pl.ds(C, 16)] = zf
            o_v[p, i, pl.ds(C + 16, 16)] = zf
        pltpu.async_copy(o_v.at[p],
                         out_hbm.at[pl.ds(wid * SW + t * NB, NB), :], semo)

    phase1(0, 0)

    def pair_body(q, carry):
        t_a = 2 * q
        phase1(t_a + 1, 1)
        drain_gathers(0)

        @pl.when(q > 0)
        def _():
            drain_out(0)
        compute(t_a, 0)

        @pl.when(q < _NPAIR - 1)
        def _():
            phase1(t_a + 2, 0)
        drain_gathers(1)

        @pl.when(q > 0)
        def _():
            drain_out(1)
        compute(t_a + 1, 1)
        return carry

    lax.fori_loop(0, _NPAIR, pair_body, 0)
    drain_out(0)
    drain_out(1)


def _sc_sample(off_pairs, table):
    mesh = plsc.VectorSubcoreMesh(core_axis_name="c", subcore_axis_name="s")
    fn = functools.partial(
        pl.kernel,
        mesh=mesh,
        compiler_params=pltpu.CompilerParams(needs_layout_passes=False,
                                             use_tc_tiling_on_sc=True),
        out_type=jax.ShapeDtypeStruct((NSAMP, 128), jnp.float32),
        scratch_types=[
            pltpu.VMEM((SW * 2,), jnp.float32),
            pltpu.VMEM((2, NB), jnp.int32),
            pltpu.VMEM((2, NB), jnp.int32),
            pltpu.VMEM((2, NB), jnp.int32),
            pltpu.VMEM((2, NB), jnp.int32),
            pltpu.VMEM((2 * NB,), jnp.float32),
            pltpu.VMEM((2 * NB,), jnp.float32),
            pltpu.VMEM((2 * NB,), jnp.float32),
            pltpu.VMEM((2 * NB,), jnp.float32),
            pltpu.VMEM((2, NB, 128), jnp.float32),
            pltpu.VMEM((2, NB, 128), jnp.float32),
            pltpu.VMEM((2, NB, 128), jnp.float32),
            pltpu.VMEM((2, NB, 128), jnp.float32),
            pltpu.VMEM((2, NB, 128), jnp.float32),
            pltpu.SemaphoreType.DMA,
            pltpu.SemaphoreType.DMA,
            pltpu.SemaphoreType.DMA,
        ],
    )(_sc_sample_body)
    return fn(off_pairs, table)


_M_BLK = 512


def _mm_body(a_ref, w_ref, b_ref, o_ref):
    a_bf = a_ref[...].astype(jnp.bfloat16).reshape(_M_BLK, NO * 128)
    o_ref[...] = (jnp.dot(a_bf, w_ref[...],
                          preferred_element_type=jnp.float32) + b_ref[...])


def _matmul(a, w, b2d):
    return pl.pallas_call(
        _mm_body,
        grid=(NPIX // _M_BLK,),
        in_specs=[
            pl.BlockSpec((_M_BLK * NO, 128), lambda i: (i, 0)),
            pl.BlockSpec((NO * 128, F), lambda i: (0, 0)),
            pl.BlockSpec((1, F), lambda i: (0, 0)),
        ],
        out_specs=pl.BlockSpec((_M_BLK, F), lambda i: (i, 0)),
        out_shape=jax.ShapeDtypeStruct((NPIX, F), jnp.float32),
    )(a, w, b2d)


def kernel(inputs, W_off, b_off, W_pt, b_pt):
    offs, table = _off_conv(inputs, W_off, b_off.reshape(1, 2 * NO))
    off_pairs = offs.reshape(NSAMP * 2)
    sampled = _sc_sample(off_pairs, table)
    wp = jnp.zeros((NO, 128, F), jnp.float32)
    wp = wp.at[:, :C, :].set(W_pt.reshape(NO, C, F))
    w = wp.reshape(NO * 128, F).astype(jnp.bfloat16)
    out = _matmul(sampled, w, b_pt.reshape(1, F))
    return out.reshape(1, H, W, F)
